# baseline (speedup 1.0000x reference)
import jax
import jax.numpy as jnp
from jax import lax
from jax.experimental import pallas as pl
from jax.experimental.pallas import tpu as pltpu

B, N, D, E = 4, 2048, 4096, 64
CAP = 40
BN = 256
NB = N // BN
EPS = 1e-9


def _fused_kernel(x_ref, w_ref, comb_ref, disp_ref, loss_ref, z_ref,
                  pbuf, cntbuf, c1_ref, c2_ref, pacc_ref, zacc_ref):
    b = pl.program_id(0)
    nb = pl.program_id(1)

    @pl.when((b == 0) & (nb == 0))
    def _init_scalars():
        loss_ref[...] = jnp.zeros_like(loss_ref)
        z_ref[...] = jnp.zeros_like(z_ref)

    @pl.when(b < B)
    def _route():
        @pl.when(nb == 0)
        def _init():
            c1_ref[...] = jnp.zeros_like(c1_ref)
            c2_ref[...] = jnp.zeros_like(c2_ref)
            pacc_ref[...] = jnp.zeros_like(pacc_ref)
            zacc_ref[...] = jnp.zeros_like(zacc_ref)

        x = x_ref[0]
        w = w_ref[...]
        logits = jnp.dot(x, w, preferred_element_type=jnp.float32)
        m = jnp.max(logits, axis=1, keepdims=True)
        ex = jnp.exp(logits - m)
        s = jnp.sum(ex, axis=1, keepdims=True)
        raw = ex / s
        ii = lax.broadcasted_iota(jnp.int32, (BN, E), 1)
        g1 = jnp.max(raw, axis=1, keepdims=True)
        i1 = jnp.min(jnp.where(raw == g1, ii, E), axis=1, keepdims=True)
        m1 = (ii == i1).astype(jnp.float32)
        raw2 = raw * (1.0 - m1)
        g2 = jnp.max(raw2, axis=1, keepdims=True)
        i2 = jnp.min(jnp.where(raw2 == g2, ii, E), axis=1, keepdims=True)
        m2 = (ii == i2).astype(jnp.float32)
        denom = g1 + g2 + EPS
        g1n = g1 / denom
        g2n = g2 / denom
        ti = lax.broadcasted_iota(jnp.int32, (BN, BN), 0)
        tj = lax.broadcasted_iota(jnp.int32, (BN, BN), 1)
        ltri = (tj < ti).astype(jnp.float32)
        ex1 = jnp.dot(ltri, m1, preferred_element_type=jnp.float32)
        ex2 = jnp.dot(ltri, m2, preferred_element_type=jnp.float32)
        pos1 = jnp.sum((ex1 + c1_ref[...]) * m1, axis=1, keepdims=True)
        rp2 = jnp.sum((ex2 + c2_ref[...]) * m2, axis=1, keepdims=True)
        c1_ref[...] += jnp.sum(m1, axis=0, keepdims=True)
        c2_ref[...] += jnp.sum(m2, axis=0, keepdims=True)
        pacc_ref[...] += jnp.sum(raw, axis=0, keepdims=True)
        zacc_ref[...] += jnp.sum(m + jnp.log(s), axis=0, keepdims=True)
        t1 = jnp.where(pos1 < float(CAP),
                       i1.astype(jnp.float32) * float(CAP) + pos1, -1.0)
        zero = jnp.zeros_like(g1)
        par = b % 2
        pbuf[par, pl.ds(nb * BN, BN), :] = jnp.concatenate(
            [g1n, g2n, t1, i2.astype(jnp.float32), rp2, zero, zero, zero],
            axis=1)

        @pl.when(nb == NB - 1)
        def _finalize():
            cntbuf[par] = c1_ref[...]
            loss_ref[...] += jnp.sum(pacc_ref[...] * c1_ref[...], axis=1,
                                     keepdims=True) * (float(E) / (B * float(N) * float(N)))
            z_ref[...] += zacc_ref[...] * (1.0 / B)

    @pl.when(b >= 1)
    def _materialize():
        par = (b - 1) % 2
        pb = pbuf[par, pl.ds(nb * BN, BN), :]
        g1 = pb[:, 0:1]
        g2 = pb[:, 1:2]
        t1 = pb[:, 2:3].astype(jnp.int32)
        i2 = pb[:, 3:4].astype(jnp.int32)
        rp2 = pb[:, 4:5]
        cnt = jnp.minimum(cntbuf[par], float(CAP))
        ii = lax.broadcasted_iota(jnp.int32, (BN, E), 1)
        m2 = (ii == i2).astype(jnp.float32)
        pos2 = rp2 + jnp.sum(m2 * cnt, axis=1, keepdims=True)
        t2 = jnp.where(pos2 < float(CAP),
                       i2.astype(jnp.float32) * float(CAP) + pos2,
                       -1.0).astype(jnp.int32)
        k2 = lax.broadcasted_iota(jnp.int32, (BN, E * CAP), 1)
        comb = jnp.where(k2 == t1, g1, 0.0) + jnp.where(k2 == t2, g2, 0.0)
        comb_ref[0] = comb
        disp_ref[0] = jnp.where(comb != 0.0, 1.0, 0.0)


def kernel(x, w_gating):
    comb, disp, loss, z = pl.pallas_call(
        _fused_kernel,
        grid=(B + 1, NB),
        in_specs=[
            pl.BlockSpec((1, BN, D),
                         lambda b, nb: (jnp.where(b < B, b, B - 1),
                                        jnp.where(b < B, nb, NB - 1), 0)),
            pl.BlockSpec((D, E), lambda b, nb: (0, 0)),
        ],
        out_specs=[
            pl.BlockSpec((1, BN, E * CAP),
                         lambda b, nb: (jnp.where(b >= 1, b - 1, 0),
                                        jnp.where(b >= 1, nb, 0), 0)),
            pl.BlockSpec((1, BN, E * CAP),
                         lambda b, nb: (jnp.where(b >= 1, b - 1, 0),
                                        jnp.where(b >= 1, nb, 0), 0)),
            pl.BlockSpec((1, 1), lambda b, nb: (0, 0)),
            pl.BlockSpec((1, 1), lambda b, nb: (0, 0)),
        ],
        out_shape=[
            jax.ShapeDtypeStruct((B, N, E * CAP), jnp.float32),
            jax.ShapeDtypeStruct((B, N, E * CAP), jnp.float32),
            jax.ShapeDtypeStruct((1, 1), jnp.float32),
            jax.ShapeDtypeStruct((1, 1), jnp.float32),
        ],
        scratch_shapes=[
            pltpu.VMEM((2, N, 8), jnp.float32),
            pltpu.VMEM((2, 1, E), jnp.float32),
            pltpu.VMEM((1, E), jnp.float32),
            pltpu.VMEM((1, E), jnp.float32),
            pltpu.VMEM((1, E), jnp.float32),
            pltpu.VMEM((1, 1), jnp.float32),
        ],
    )(x, w_gating)
    return (disp.reshape(B, N, E, CAP), comb.reshape(B, N, E, CAP),
            loss.reshape(()), z.reshape(()))


# speedup vs baseline: 1.0442x; 1.0442x over previous
import jax
import jax.numpy as jnp
from jax import lax
from jax.experimental import pallas as pl
from jax.experimental.pallas import tpu as pltpu

B, N, D, E = 4, 2048, 4096, 64
CAP = 40
BN = 512
NB = N // BN
EPS = 1e-9


def _fused_kernel(x_ref, w_ref, comb_ref, disp_ref, loss_ref, z_ref,
                  pbuf, cntbuf, c1_ref, c2_ref, pacc_ref, zacc_ref):
    b = pl.program_id(0)
    nb = pl.program_id(1)

    @pl.when((b == 0) & (nb == 0))
    def _init_scalars():
        loss_ref[...] = jnp.zeros_like(loss_ref)
        z_ref[...] = jnp.zeros_like(z_ref)

    @pl.when(b < B)
    def _route():
        @pl.when(nb == 0)
        def _init():
            c1_ref[...] = jnp.zeros_like(c1_ref)
            c2_ref[...] = jnp.zeros_like(c2_ref)
            pacc_ref[...] = jnp.zeros_like(pacc_ref)
            zacc_ref[...] = jnp.zeros_like(zacc_ref)

        x = x_ref[0]
        w = w_ref[...]
        logits = jnp.dot(x, w, preferred_element_type=jnp.float32)
        m = jnp.max(logits, axis=1, keepdims=True)
        ex = jnp.exp(logits - m)
        s = jnp.sum(ex, axis=1, keepdims=True)
        raw = ex / s
        ii = lax.broadcasted_iota(jnp.int32, (BN, E), 1)
        g1 = jnp.max(raw, axis=1, keepdims=True)
        i1 = jnp.min(jnp.where(raw == g1, ii, E), axis=1, keepdims=True)
        m1 = (ii == i1).astype(jnp.float32)
        raw2 = raw * (1.0 - m1)
        g2 = jnp.max(raw2, axis=1, keepdims=True)
        i2 = jnp.min(jnp.where(raw2 == g2, ii, E), axis=1, keepdims=True)
        m2 = (ii == i2).astype(jnp.float32)
        denom = g1 + g2 + EPS
        g1n = g1 / denom
        g2n = g2 / denom
        ti = lax.broadcasted_iota(jnp.int32, (BN, BN), 0)
        tj = lax.broadcasted_iota(jnp.int32, (BN, BN), 1)
        ltri = (tj < ti).astype(jnp.float32)
        ex1 = jnp.dot(ltri, m1, preferred_element_type=jnp.float32)
        ex2 = jnp.dot(ltri, m2, preferred_element_type=jnp.float32)
        pos1 = jnp.sum((ex1 + c1_ref[...]) * m1, axis=1, keepdims=True)
        rp2 = jnp.sum((ex2 + c2_ref[...]) * m2, axis=1, keepdims=True)
        c1_ref[...] += jnp.sum(m1, axis=0, keepdims=True)
        c2_ref[...] += jnp.sum(m2, axis=0, keepdims=True)
        pacc_ref[...] += jnp.sum(raw, axis=0, keepdims=True)
        zacc_ref[...] += jnp.sum(m + jnp.log(s), axis=0, keepdims=True)
        t1 = jnp.where(pos1 < float(CAP),
                       i1.astype(jnp.float32) * float(CAP) + pos1, -1.0)
        zero = jnp.zeros_like(g1)
        par = b % 2
        pbuf[par, pl.ds(nb * BN, BN), :] = jnp.concatenate(
            [g1n, g2n, t1, i2.astype(jnp.float32), rp2, zero, zero, zero],
            axis=1)

        @pl.when(nb == NB - 1)
        def _finalize():
            cntbuf[par] = c1_ref[...]
            loss_ref[...] += jnp.sum(pacc_ref[...] * c1_ref[...], axis=1,
                                     keepdims=True) * (float(E) / (B * float(N) * float(N)))
            z_ref[...] += zacc_ref[...] * (1.0 / B)

    @pl.when(b >= 1)
    def _materialize():
        par = (b - 1) % 2
        pb = pbuf[par, pl.ds(nb * BN, BN), :]
        g1 = pb[:, 0:1]
        g2 = pb[:, 1:2]
        t1 = pb[:, 2:3].astype(jnp.int32)
        i2 = pb[:, 3:4].astype(jnp.int32)
        rp2 = pb[:, 4:5]
        cnt = jnp.minimum(cntbuf[par], float(CAP))
        ii = lax.broadcasted_iota(jnp.int32, (BN, E), 1)
        m2 = (ii == i2).astype(jnp.float32)
        pos2 = rp2 + jnp.sum(m2 * cnt, axis=1, keepdims=True)
        t2 = jnp.where(pos2 < float(CAP),
                       i2.astype(jnp.float32) * float(CAP) + pos2,
                       -1.0).astype(jnp.int32)
        k2 = lax.broadcasted_iota(jnp.int32, (BN, E * CAP), 1)
        comb = jnp.where(k2 == t1, g1, 0.0) + jnp.where(k2 == t2, g2, 0.0)
        comb_ref[0] = comb
        disp_ref[0] = jnp.where(comb != 0.0, 1.0, 0.0)


def kernel(x, w_gating):
    comb, disp, loss, z = pl.pallas_call(
        _fused_kernel,
        grid=(B + 1, NB),
        in_specs=[
            pl.BlockSpec((1, BN, D),
                         lambda b, nb: (jnp.where(b < B, b, B - 1),
                                        jnp.where(b < B, nb, NB - 1), 0)),
            pl.BlockSpec((D, E), lambda b, nb: (0, 0)),
        ],
        out_specs=[
            pl.BlockSpec((1, BN, E * CAP),
                         lambda b, nb: (jnp.where(b >= 1, b - 1, 0),
                                        jnp.where(b >= 1, nb, 0), 0)),
            pl.BlockSpec((1, BN, E * CAP),
                         lambda b, nb: (jnp.where(b >= 1, b - 1, 0),
                                        jnp.where(b >= 1, nb, 0), 0)),
            pl.BlockSpec((1, 1), lambda b, nb: (0, 0)),
            pl.BlockSpec((1, 1), lambda b, nb: (0, 0)),
        ],
        out_shape=[
            jax.ShapeDtypeStruct((B, N, E * CAP), jnp.float32),
            jax.ShapeDtypeStruct((B, N, E * CAP), jnp.float32),
            jax.ShapeDtypeStruct((1, 1), jnp.float32),
            jax.ShapeDtypeStruct((1, 1), jnp.float32),
        ],
        scratch_shapes=[
            pltpu.VMEM((2, N, 8), jnp.float32),
            pltpu.VMEM((2, 1, E), jnp.float32),
            pltpu.VMEM((1, E), jnp.float32),
            pltpu.VMEM((1, E), jnp.float32),
            pltpu.VMEM((1, E), jnp.float32),
            pltpu.VMEM((1, 1), jnp.float32),
        ],
    )(x, w_gating)
    return (disp.reshape(B, N, E, CAP), comb.reshape(B, N, E, CAP),
            loss.reshape(()), z.reshape(()))
